# baseline (speedup 1.0000x reference)
import jax
import jax.numpy as jnp
from jax.experimental import pallas as pl
from jax.experimental.pallas import tpu as pltpu

_B, _N1, _N2, _D = 4, 4096, 4096, 3
_DP = 8
_BLK1 = 512
_NB1 = _N1 // _BLK1


def _chamfer_body(x1_ref, x2_ref, s1_ref, s2_ref, cm_ref):
    i = pl.program_id(1)
    a = x1_ref[0]
    b = x2_ref[0]
    inner = jax.lax.dot_general(
        a, b, (((0,), (0,)), ((), ())),
        preferred_element_type=jnp.float32)
    sq1 = jnp.sum(a * a, axis=0)
    sq2 = jnp.sum(b * b, axis=0)
    e = sq2[None, :] - 2.0 * inner
    f = sq1[:, None] - 2.0 * inner
    rowsum = jnp.sum(jnp.maximum(jnp.min(e, axis=1) + sq1, 0.0))
    colmin = jnp.min(f, axis=0)[None, :]

    @pl.when(i == 0)
    def _():
        s1_ref[...] = rowsum.reshape(1, 1, 1)
        cm_ref[...] = colmin

    @pl.when(i > 0)
    def _():
        s1_ref[...] += rowsum.reshape(1, 1, 1)
        cm_ref[...] = jnp.minimum(cm_ref[...], colmin)

    @pl.when(i == _NB1 - 1)
    def _():
        s2_ref[...] = jnp.sum(
            jnp.maximum(cm_ref[...] + sq2[None, :], 0.0)).reshape(1, 1, 1)


def kernel(xyz1, xyz2):
    x1t = jnp.pad(jnp.moveaxis(xyz1, -1, -2), ((0, 0), (0, _DP - _D), (0, 0)))
    x2t = jnp.pad(jnp.moveaxis(xyz2, -1, -2), ((0, 0), (0, _DP - _D), (0, 0)))

    s1, s2 = pl.pallas_call(
        _chamfer_body,
        grid=(_B, _NB1),
        in_specs=[
            pl.BlockSpec((1, _DP, _BLK1), lambda b, i: (b, 0, i)),
            pl.BlockSpec((1, _DP, _N2), lambda b, i: (b, 0, 0)),
        ],
        out_specs=[
            pl.BlockSpec((1, 1, 1), lambda b, i: (b, 0, 0)),
            pl.BlockSpec((1, 1, 1), lambda b, i: (b, 0, 0)),
        ],
        out_shape=[
            jax.ShapeDtypeStruct((_B, 1, 1), jnp.float32),
            jax.ShapeDtypeStruct((_B, 1, 1), jnp.float32),
        ],
        scratch_shapes=[pltpu.VMEM((1, _N2), jnp.float32)],
        compiler_params=pltpu.CompilerParams(
            dimension_semantics=("parallel", "arbitrary")),
    )(x1t, x2t)

    return jnp.sum(s1) / (_B * _N1), jnp.sum(s2) / (_B * _N2)


# speedup vs baseline: 1.0637x; 1.0637x over previous
import jax
import jax.numpy as jnp
from jax.experimental import pallas as pl
from jax.experimental.pallas import tpu as pltpu

_B, _N1, _N2, _D = 4, 4096, 4096, 3
_DP = 8
_BLK1 = 512
_NB1 = _N1 // _BLK1


def _chamfer_body(x1_ref, x2_ref, s1_ref, s2_ref, cm_ref):
    i = pl.program_id(1)
    a = x1_ref[0]
    b = x2_ref[0]
    inner2 = jax.lax.dot_general(
        -2.0 * a, b, (((0,), (0,)), ((), ())),
        preferred_element_type=jnp.float32)
    sq1 = jnp.sum(a * a, axis=0)
    sq2 = jnp.sum(b * b, axis=0)
    e = sq2[None, :] + inner2
    f = sq1[:, None] + inner2
    rowsum = jnp.sum(jnp.maximum(jnp.min(e, axis=1) + sq1, 0.0))
    colmin = jnp.min(f, axis=0)[None, :]

    @pl.when(i == 0)
    def _():
        s1_ref[...] = rowsum.reshape(1, 1, 1)
        cm_ref[...] = colmin

    @pl.when(i > 0)
    def _():
        s1_ref[...] += rowsum.reshape(1, 1, 1)
        cm_ref[...] = jnp.minimum(cm_ref[...], colmin)

    @pl.when(i == _NB1 - 1)
    def _():
        s2_ref[...] = jnp.sum(
            jnp.maximum(cm_ref[...] + sq2[None, :], 0.0)).reshape(1, 1, 1)


def kernel(xyz1, xyz2):
    x1t = jnp.pad(jnp.moveaxis(xyz1, -1, -2), ((0, 0), (0, _DP - _D), (0, 0)))
    x2t = jnp.pad(jnp.moveaxis(xyz2, -1, -2), ((0, 0), (0, _DP - _D), (0, 0)))

    s1, s2 = pl.pallas_call(
        _chamfer_body,
        grid=(_B, _NB1),
        in_specs=[
            pl.BlockSpec((1, _DP, _BLK1), lambda b, i: (b, 0, i)),
            pl.BlockSpec((1, _DP, _N2), lambda b, i: (b, 0, 0)),
        ],
        out_specs=[
            pl.BlockSpec((1, 1, 1), lambda b, i: (b, 0, 0)),
            pl.BlockSpec((1, 1, 1), lambda b, i: (b, 0, 0)),
        ],
        out_shape=[
            jax.ShapeDtypeStruct((_B, 1, 1), jnp.float32),
            jax.ShapeDtypeStruct((_B, 1, 1), jnp.float32),
        ],
        scratch_shapes=[pltpu.VMEM((1, _N2), jnp.float32)],
        compiler_params=pltpu.CompilerParams(
            dimension_semantics=("parallel", "arbitrary")),
    )(x1t, x2t)

    return jnp.sum(s1) / (_B * _N1), jnp.sum(s2) / (_B * _N2)


# speedup vs baseline: 1.0883x; 1.0231x over previous
import jax
import jax.numpy as jnp
from jax.experimental import pallas as pl
from jax.experimental.pallas import tpu as pltpu

_B, _N1, _N2, _D = 4, 4096, 4096, 3
_DP = 8
_BLK1 = 1024
_NB1 = _N1 // _BLK1


def _chamfer_body(x1_ref, x2_ref, s1_ref, s2_ref, cm_ref):
    i = pl.program_id(1)
    a = x1_ref[0]
    b = x2_ref[0]
    inner2 = jax.lax.dot_general(
        -2.0 * a, b, (((0,), (0,)), ((), ())),
        preferred_element_type=jnp.float32)
    sq1 = jnp.sum(a * a, axis=0)
    sq2 = jnp.sum(b * b, axis=0)
    e = sq2[None, :] + inner2
    f = sq1[:, None] + inner2
    rowsum = jnp.sum(jnp.maximum(jnp.min(e, axis=1) + sq1, 0.0))
    colmin = jnp.min(f, axis=0)[None, :]

    @pl.when(i == 0)
    def _():
        s1_ref[...] = rowsum.reshape(1, 1, 1)
        cm_ref[...] = colmin

    @pl.when(i > 0)
    def _():
        s1_ref[...] += rowsum.reshape(1, 1, 1)
        cm_ref[...] = jnp.minimum(cm_ref[...], colmin)

    @pl.when(i == _NB1 - 1)
    def _():
        s2_ref[...] = jnp.sum(
            jnp.maximum(cm_ref[...] + sq2[None, :], 0.0)).reshape(1, 1, 1)


def kernel(xyz1, xyz2):
    x1t = jnp.pad(jnp.moveaxis(xyz1, -1, -2), ((0, 0), (0, _DP - _D), (0, 0)))
    x2t = jnp.pad(jnp.moveaxis(xyz2, -1, -2), ((0, 0), (0, _DP - _D), (0, 0)))

    s1, s2 = pl.pallas_call(
        _chamfer_body,
        grid=(_B, _NB1),
        in_specs=[
            pl.BlockSpec((1, _DP, _BLK1), lambda b, i: (b, 0, i)),
            pl.BlockSpec((1, _DP, _N2), lambda b, i: (b, 0, 0)),
        ],
        out_specs=[
            pl.BlockSpec((1, 1, 1), lambda b, i: (b, 0, 0)),
            pl.BlockSpec((1, 1, 1), lambda b, i: (b, 0, 0)),
        ],
        out_shape=[
            jax.ShapeDtypeStruct((_B, 1, 1), jnp.float32),
            jax.ShapeDtypeStruct((_B, 1, 1), jnp.float32),
        ],
        scratch_shapes=[pltpu.VMEM((1, _N2), jnp.float32)],
        compiler_params=pltpu.CompilerParams(
            dimension_semantics=("parallel", "arbitrary")),
    )(x1t, x2t)

    return jnp.sum(s1) / (_B * _N1), jnp.sum(s2) / (_B * _N2)


# speedup vs baseline: 1.3627x; 1.2521x over previous
import functools

import jax
import jax.numpy as jnp
from jax.experimental import pallas as pl
from jax.experimental.pallas import tpu as pltpu

_B, _N1, _N2, _D = 4, 4096, 4096, 3
_DP = 8
_BLK1 = 1024
_CH = 1024
_NCH = _N2 // _CH
_NB1 = _N1 // _BLK1


def _chamfer_body(x1_ref, x2_ref, s1_ref, s2_ref, cm_ref):
    i = pl.program_id(1)
    a = x1_ref[0]
    b = x2_ref[0]
    a2 = -2.0 * a
    sq1 = jnp.sum(a * a, axis=0)
    sq2 = jnp.sum(b * b, axis=0)
    inner2 = jax.lax.dot_general(
        a2, b, (((0,), (0,)), ((), ())),
        preferred_element_type=jnp.float32)
    e = sq2[None, :] + inner2
    f = sq1[:, None] + inner2
    v = functools.reduce(
        jnp.minimum, [e[:, j * 128:(j + 1) * 128] for j in range(_N2 // 128)])
    rowmin = jnp.min(v.T, axis=0)
    rowsum = jnp.sum(jnp.maximum(rowmin + sq1, 0.0))
    colmin = jnp.min(f, axis=0)[None, :]

    @pl.when(i == 0)
    def _():
        s1_ref[...] = rowsum.reshape(1, 1, 1)
        cm_ref[...] = colmin

    @pl.when(i > 0)
    def _():
        s1_ref[...] += rowsum.reshape(1, 1, 1)
        cm_ref[...] = jnp.minimum(cm_ref[...], colmin)

    @pl.when(i == _NB1 - 1)
    def _():
        s2_ref[...] = jnp.sum(
            jnp.maximum(cm_ref[...] + sq2[None, :], 0.0)).reshape(1, 1, 1)


def kernel(xyz1, xyz2):
    x1t = jnp.pad(jnp.moveaxis(xyz1, -1, -2), ((0, 0), (0, _DP - _D), (0, 0)))
    x2t = jnp.pad(jnp.moveaxis(xyz2, -1, -2), ((0, 0), (0, _DP - _D), (0, 0)))

    s1, s2 = pl.pallas_call(
        _chamfer_body,
        grid=(_B, _NB1),
        in_specs=[
            pl.BlockSpec((1, _DP, _BLK1), lambda b, i: (b, 0, i)),
            pl.BlockSpec((1, _DP, _N2), lambda b, i: (b, 0, 0)),
        ],
        out_specs=[
            pl.BlockSpec((1, 1, 1), lambda b, i: (b, 0, 0)),
            pl.BlockSpec((1, 1, 1), lambda b, i: (b, 0, 0)),
        ],
        out_shape=[
            jax.ShapeDtypeStruct((_B, 1, 1), jnp.float32),
            jax.ShapeDtypeStruct((_B, 1, 1), jnp.float32),
        ],
        scratch_shapes=[pltpu.VMEM((1, _N2), jnp.float32)],
        compiler_params=pltpu.CompilerParams(
            dimension_semantics=("parallel", "arbitrary")),
    )(x1t, x2t)

    return jnp.sum(s1) / (_B * _N1), jnp.sum(s2) / (_B * _N2)


# speedup vs baseline: 1.4777x; 1.0844x over previous
import functools

import jax
import jax.numpy as jnp
from jax.experimental import pallas as pl
from jax.experimental.pallas import tpu as pltpu

_B, _N1, _N2, _D = 4, 4096, 4096, 3
_DP = 8
_BLK = 512
_NB = _N1 // _BLK


def _rowmin_lanes(e):
    v = functools.reduce(
        jnp.minimum, [e[:, j * 128:(j + 1) * 128] for j in range(_N2 // 128)])
    return jnp.min(v.T, axis=0)


def _chamfer_body(x1_ref, x2_ref, s1_ref, s2_ref):
    for b in range(1):
        xb1 = x1_ref[0]
        xb2 = x2_ref[0]
        a2f = -2.0 * xb1
        sq1f = jnp.sum(xb1 * xb1, axis=0)
        sq2 = jnp.sum(xb2 * xb2, axis=0)
        rowsums = []
        cm = None
        for k in range(_NB):
            lo = k * _BLK
            a2 = a2f[:, lo:lo + _BLK]
            inner2 = jax.lax.dot_general(
                a2, xb2, (((0,), (0,)), ((), ())),
                preferred_element_type=jnp.float32)
            e = sq2[None, :] + inner2
            f = sq1f[lo:lo + _BLK][:, None] + inner2
            rowmin = _rowmin_lanes(e) + sq1f[lo:lo + _BLK]
            rowsums.append(jnp.sum(jnp.maximum(rowmin, 0.0)))
            cmk = jnp.min(f, axis=0)
            cm = cmk if cm is None else jnp.minimum(cm, cmk)
        s1_ref[...] = sum(rowsums).reshape(1, 1, 1)
        s2_ref[...] = jnp.sum(
            jnp.maximum(cm + sq2, 0.0)).reshape(1, 1, 1)


def kernel(xyz1, xyz2):
    x1t = jnp.pad(jnp.moveaxis(xyz1, -1, -2), ((0, 0), (0, _DP - _D), (0, 0)))
    x2t = jnp.pad(jnp.moveaxis(xyz2, -1, -2), ((0, 0), (0, _DP - _D), (0, 0)))

    s1, s2 = pl.pallas_call(
        _chamfer_body,
        grid=(_B,),
        in_specs=[
            pl.BlockSpec((1, _DP, _N1), lambda i: (i, 0, 0)),
            pl.BlockSpec((1, _DP, _N2), lambda i: (i, 0, 0)),
        ],
        out_specs=[
            pl.BlockSpec((1, 1, 1), lambda i: (i, 0, 0)),
            pl.BlockSpec((1, 1, 1), lambda i: (i, 0, 0)),
        ],
        out_shape=[
            jax.ShapeDtypeStruct((_B, 1, 1), jnp.float32),
            jax.ShapeDtypeStruct((_B, 1, 1), jnp.float32),
        ],
    )(x1t, x2t)

    return jnp.sum(s1) / (_B * _N1), jnp.sum(s2) / (_B * _N2)


# speedup vs baseline: 1.5136x; 1.0243x over previous
import functools

import jax
import jax.numpy as jnp
from jax.experimental import pallas as pl
from jax.experimental.pallas import tpu as pltpu

_B, _N1, _N2, _D = 4, 4096, 4096, 3
_DP = 8
_BLK = 512
_NB = _N1 // _BLK


def _rowmin_lanes(e):
    v = functools.reduce(
        jnp.minimum, [e[:, j * 128:(j + 1) * 128] for j in range(_N2 // 128)])
    return jnp.min(v.T, axis=0)


def _chamfer_body(x1_ref, x2_ref, s1_ref, s2_ref):
    for b in range(1):
        xb1 = x1_ref[0]
        xb2 = x2_ref[0]
        a2f = -2.0 * xb1
        sq1f = jnp.sum(xb1 * xb1, axis=0)
        sq2 = jnp.sum(xb2 * xb2, axis=0)
        rowsums = []
        cm = None
        for k in range(_NB):
            lo = k * _BLK
            a2 = a2f[:, lo:lo + _BLK]
            sq1b = sq1f[lo:lo + _BLK]
            cmks, vs = [], []
            for h in range(2):
                hl = h * (_N2 // 2)
                inner2 = jax.lax.dot_general(
                    a2, xb2[:, hl:hl + _N2 // 2], (((0,), (0,)), ((), ())),
                    preferred_element_type=jnp.float32)
                e = sq2[None, hl:hl + _N2 // 2] + inner2
                f = sq1b[:, None] + inner2
                vs.append(functools.reduce(
                    jnp.minimum,
                    [e[:, j * 128:(j + 1) * 128] for j in range(_N2 // 256)]))
                cmks.append(jnp.min(f, axis=0))
            rowmin = jnp.min(
                jnp.minimum(vs[0], vs[1]).T, axis=0) + sq1b
            rowsums.append(jnp.sum(jnp.maximum(rowmin, 0.0)))
            cmk = jnp.concatenate(cmks)
            cm = cmk if cm is None else jnp.minimum(cm, cmk)
        s1_ref[...] = sum(rowsums).reshape(1, 1, 1)
        s2_ref[...] = jnp.sum(
            jnp.maximum(cm + sq2, 0.0)).reshape(1, 1, 1)


def kernel(xyz1, xyz2):
    x1t = jnp.pad(jnp.moveaxis(xyz1, -1, -2), ((0, 0), (0, _DP - _D), (0, 0)))
    x2t = jnp.pad(jnp.moveaxis(xyz2, -1, -2), ((0, 0), (0, _DP - _D), (0, 0)))

    s1, s2 = pl.pallas_call(
        _chamfer_body,
        grid=(_B,),
        in_specs=[
            pl.BlockSpec((1, _DP, _N1), lambda i: (i, 0, 0)),
            pl.BlockSpec((1, _DP, _N2), lambda i: (i, 0, 0)),
        ],
        out_specs=[
            pl.BlockSpec((1, 1, 1), lambda i: (i, 0, 0)),
            pl.BlockSpec((1, 1, 1), lambda i: (i, 0, 0)),
        ],
        out_shape=[
            jax.ShapeDtypeStruct((_B, 1, 1), jnp.float32),
            jax.ShapeDtypeStruct((_B, 1, 1), jnp.float32),
        ],
    )(x1t, x2t)

    return jnp.sum(s1) / (_B * _N1), jnp.sum(s2) / (_B * _N2)
